# Initial kernel scaffold; baseline (speedup 1.0000x reference)
import jax
import jax.numpy as jnp
from jax.experimental import pallas as pl


def kernel(seg, confidence, contrast_logits, contrast_target, target, ln_gamma, ln_beta):
    raise NotImplementedError("write your pallas kernel here")


# speedup vs baseline: 2.9808x; 2.9808x over previous
import jax
import jax.numpy as jnp
from jax.experimental import pallas as pl
from jax.experimental.pallas import tpu as pltpu

_NC = 19
_CM = 95
_IGNORE = -1
_PPC_W = 0.01
_PPD_W = 0.001
_UNCER_W = 1.0

_STEPS = 64
_ROWS = 131072 // _STEPS
_HCHUNK = 128 // (_STEPS // 8)


def _body(seg_ref, tgt_ref, conf_ref, x_ref, ct_ref, g_ref, bb_ref, out_ref):
    i = pl.program_id(0)

    seg = seg_ref[0]
    tgt = tgt_ref[0]
    conf = conf_ref[0]
    valid = tgt != _IGNORE
    vf = valid.astype(jnp.float32)
    tc = jnp.clip(tgt, 0, _NC - 1)

    m = jnp.max(seg, axis=0)
    cls_iota = jax.lax.broadcasted_iota(jnp.int32, seg.shape, 0)
    amax = jnp.min(jnp.where(seg == m[None], cls_iota, _NC), axis=0)
    seg2 = jnp.where(cls_iota == amax[None], -jnp.inf, seg)
    m2 = jnp.max(seg2, axis=0)

    s = jnp.sum(jnp.exp(seg - m[None]), axis=0)
    lse = m + jnp.log(s)
    seg_t = jnp.sum(jnp.where(cls_iota == tc[None], seg, 0.0), axis=0)
    nll_sum = jnp.sum((lse - seg_t) * vf)

    label = amax == tgt
    p = 1.0 / (1.0 + jnp.exp(m2 - m))
    u = jnp.where(label, 1.0 - p, p)
    bce = jnp.maximum(conf, 0.0) - conf * u + jnp.log1p(jnp.exp(-jnp.abs(conf)))
    bce_sum = jnp.sum(bce * vf)
    cnt = jnp.sum(vf)

    x = x_ref[...]
    ct = ct_ref[0, 0, :]
    cvalid = ct != _IGNORE
    cvf = cvalid.astype(jnp.float32)
    ctc = jnp.clip(ct, 0, _CM - 1)

    mu = jnp.mean(x, axis=1, keepdims=True)
    xc = x - mu
    var = jnp.mean(xc * xc, axis=1, keepdims=True)
    rs = jax.lax.rsqrt(var + 1e-5)
    normed = xc * rs * g_ref[0][None, :] + bb_ref[0][None, :]
    mx = jnp.max(normed, axis=1, keepdims=True)
    s2 = jnp.sum(jnp.exp(normed - mx), axis=1)
    lse2 = mx[:, 0] + jnp.log(s2)

    lane = jax.lax.broadcasted_iota(jnp.int32, (_ROWS, _CM), 1)
    oh = lane == ctc[:, None]
    nsel = jnp.sum(jnp.where(oh, normed, 0.0), axis=1)
    xsel = jnp.sum(jnp.where(oh, x, 0.0), axis=1)

    nll2_sum = jnp.sum((lse2 - nsel) * cvf)
    ppd_sum = jnp.sum((1.0 - xsel) ** 2 * cvf)
    ccnt = jnp.sum(cvf)

    @pl.when(i == 0)
    def _():
        out_ref[0] = 0.0
        out_ref[1] = 0.0
        out_ref[2] = 0.0
        out_ref[3] = 0.0
        out_ref[4] = 0.0
        out_ref[5] = 0.0

    out_ref[0] += nll_sum
    out_ref[1] += bce_sum
    out_ref[2] += cnt
    out_ref[3] += nll2_sum
    out_ref[4] += ppd_sum
    out_ref[5] += ccnt


def kernel(seg, confidence, contrast_logits, contrast_target, target, ln_gamma, ln_beta):
    n = contrast_target.shape[0]
    ct3 = contrast_target.reshape(_STEPS, 1, n // _STEPS)
    g2 = ln_gamma.reshape(1, _CM)
    bb2 = ln_beta.reshape(1, _CM)

    sums = pl.pallas_call(
        _body,
        grid=(_STEPS,),
        in_specs=[
            pl.BlockSpec((1, _NC, _HCHUNK, 128), lambda i: (i // 8, 0, i % 8, 0)),
            pl.BlockSpec((1, _HCHUNK, 128), lambda i: (i // 8, i % 8, 0)),
            pl.BlockSpec((1, _HCHUNK, 128), lambda i: (i // 8, i % 8, 0)),
            pl.BlockSpec((_ROWS, _CM), lambda i: (i, 0)),
            pl.BlockSpec((1, 1, _ROWS), lambda i: (i, 0, 0)),
            pl.BlockSpec((1, _CM), lambda i: (0, 0)),
            pl.BlockSpec((1, _CM), lambda i: (0, 0)),
        ],
        out_specs=pl.BlockSpec(memory_space=pltpu.SMEM),
        out_shape=jax.ShapeDtypeStruct((6,), jnp.float32),
    )(seg, target, confidence, contrast_logits, ct3, g2, bb2)

    nll_sum, bce_sum, cnt, nll2_sum, ppd_sum, ccnt = (
        sums[0], sums[1], sums[2], sums[3], sums[4], sums[5])
    seg_loss = nll_sum / jnp.maximum(cnt, 1.0)
    uncer = bce_sum / jnp.maximum(cnt, 1.0)
    ppc = nll2_sum / jnp.maximum(ccnt, 1.0)
    ppd = ppd_sum / jnp.maximum(ccnt, 1.0)
    return seg_loss + _PPC_W * ppc + _PPD_W * ppd + _UNCER_W * uncer


# speedup vs baseline: 3.7181x; 1.2474x over previous
import jax
import jax.numpy as jnp
from jax.experimental import pallas as pl
from jax.experimental.pallas import tpu as pltpu

_NC = 19
_CM = 95
_IGNORE = -1
_PPC_W = 0.01
_PPD_W = 0.001
_UNCER_W = 1.0

_STEPS = 64
_ROWS = 131072 // _STEPS
_HCHUNK = 128 // (_STEPS // 8)


def _body(seg_ref, tgt_ref, conf_ref, x_ref, ct_ref, g_ref, bb_ref, out_ref):
    i = pl.program_id(0)

    seg = seg_ref[0]
    tgt = tgt_ref[0]
    conf = conf_ref[0]
    valid = tgt != _IGNORE
    vf = valid.astype(jnp.float32)
    tc = jnp.clip(tgt, 0, _NC - 1)

    m = jnp.max(seg, axis=0)
    cls_iota = jax.lax.broadcasted_iota(jnp.int32, seg.shape, 0)
    amax = jnp.min(jnp.where(seg == m[None], cls_iota, _NC), axis=0)
    seg2 = jnp.where(cls_iota == amax[None], -jnp.inf, seg)
    m2 = jnp.max(seg2, axis=0)

    s = jnp.sum(jnp.exp(seg - m[None]), axis=0)
    lse = m + jnp.log(s)
    seg_t = jnp.sum(jnp.where(cls_iota == tc[None], seg, 0.0), axis=0)
    nll_sum = jnp.sum((lse - seg_t) * vf)

    label = amax == tgt
    p = 1.0 / (1.0 + jnp.exp(m2 - m))
    u = jnp.where(label, 1.0 - p, p)
    bce = jnp.maximum(conf, 0.0) - conf * u + jnp.log1p(jnp.exp(-jnp.abs(conf)))
    bce_sum = jnp.sum(bce * vf)
    cnt = jnp.sum(vf)

    x = x_ref[...]
    ct = ct_ref[0, 0, :]
    cvf = (ct != _IGNORE).astype(jnp.float32)

    inv = 1.0 / _CM
    s1 = jnp.sum(x, axis=1)
    mu = s1 * inv
    s2m = jnp.sum(x * x, axis=1)
    var = s2m * inv - mu * mu
    rs = jax.lax.rsqrt(var + 1e-5)
    normed = (x - mu[:, None]) * rs[:, None] * g_ref[0][None, :] + bb_ref[0][None, :]
    s3 = jnp.sum(jnp.exp(normed), axis=1)
    lse2 = jnp.log(s3)

    lane = jax.lax.broadcasted_iota(jnp.int32, (_ROWS, _CM), 1)
    ohf = (lane == ct[:, None]).astype(jnp.float32)
    ohx = ohf * x
    sum_nsel = jnp.sum(ohf * normed)
    sum_xsel = jnp.sum(ohx)
    sum_xsel2 = jnp.sum(ohx * x)

    ccnt = jnp.sum(cvf)
    nll2_sum = jnp.sum(lse2 * cvf) - sum_nsel
    ppd_sum = ccnt - 2.0 * sum_xsel + sum_xsel2

    @pl.when(i == 0)
    def _():
        out_ref[0] = 0.0
        out_ref[1] = 0.0
        out_ref[2] = 0.0
        out_ref[3] = 0.0
        out_ref[4] = 0.0
        out_ref[5] = 0.0

    out_ref[0] += nll_sum
    out_ref[1] += bce_sum
    out_ref[2] += cnt
    out_ref[3] += nll2_sum
    out_ref[4] += ppd_sum
    out_ref[5] += ccnt


def kernel(seg, confidence, contrast_logits, contrast_target, target, ln_gamma, ln_beta):
    n = contrast_target.shape[0]
    ct3 = contrast_target.reshape(_STEPS, 1, n // _STEPS)
    g2 = ln_gamma.reshape(1, _CM)
    bb2 = ln_beta.reshape(1, _CM)

    sums = pl.pallas_call(
        _body,
        grid=(_STEPS,),
        in_specs=[
            pl.BlockSpec((1, _NC, _HCHUNK, 128), lambda i: (i // 8, 0, i % 8, 0)),
            pl.BlockSpec((1, _HCHUNK, 128), lambda i: (i // 8, i % 8, 0)),
            pl.BlockSpec((1, _HCHUNK, 128), lambda i: (i // 8, i % 8, 0)),
            pl.BlockSpec((_ROWS, _CM), lambda i: (i, 0)),
            pl.BlockSpec((1, 1, _ROWS), lambda i: (i, 0, 0)),
            pl.BlockSpec((1, _CM), lambda i: (0, 0)),
            pl.BlockSpec((1, _CM), lambda i: (0, 0)),
        ],
        out_specs=pl.BlockSpec(memory_space=pltpu.SMEM),
        out_shape=jax.ShapeDtypeStruct((6,), jnp.float32),
    )(seg, target, confidence, contrast_logits, ct3, g2, bb2)

    nll_sum, bce_sum, cnt, nll2_sum, ppd_sum, ccnt = (
        sums[0], sums[1], sums[2], sums[3], sums[4], sums[5])
    seg_loss = nll_sum / jnp.maximum(cnt, 1.0)
    uncer = bce_sum / jnp.maximum(cnt, 1.0)
    ppc = nll2_sum / jnp.maximum(ccnt, 1.0)
    ppd = ppd_sum / jnp.maximum(ccnt, 1.0)
    return seg_loss + _PPC_W * ppc + _PPD_W * ppd + _UNCER_W * uncer


# speedup vs baseline: 5.1923x; 1.3965x over previous
import jax
import jax.numpy as jnp
from jax.experimental import pallas as pl
from jax.experimental.pallas import tpu as pltpu

_NC = 19
_CM = 95
_IGNORE = -1
_PPC_W = 0.01
_PPD_W = 0.001
_UNCER_W = 1.0

_STEPS = 16
_ROWS = 131072 // _STEPS
_SPB = _STEPS // 8
_HCHUNK = 128 // _SPB


def _body(seg_ref, tgt_ref, conf_ref, x_ref, ct_ref, g_ref, bb_ref, out_ref):
    i = pl.program_id(0)

    seg = seg_ref[0]
    tgt = tgt_ref[0]
    conf = conf_ref[0]
    valid = tgt != _IGNORE
    vf = valid.astype(jnp.float32)
    tc = jnp.clip(tgt, 0, _NC - 1)

    m = jnp.max(seg, axis=0)
    cls_iota = jax.lax.broadcasted_iota(jnp.int32, seg.shape, 0)
    amax = jnp.min(jnp.where(seg == m[None], cls_iota, _NC), axis=0)
    seg2 = jnp.where(cls_iota == amax[None], -jnp.inf, seg)
    m2 = jnp.max(seg2, axis=0)

    s = jnp.sum(jnp.exp(seg - m[None]), axis=0)
    lse = m + jnp.log(s)
    seg_t = jnp.sum(jnp.where(cls_iota == tc[None], seg, 0.0), axis=0)
    nll_sum = jnp.sum((lse - seg_t) * vf)

    label = amax == tgt
    p = 1.0 / (1.0 + jnp.exp(m2 - m))
    u = jnp.where(label, 1.0 - p, p)
    bce = jnp.maximum(conf, 0.0) - conf * u + jnp.log1p(jnp.exp(-jnp.abs(conf)))
    bce_sum = jnp.sum(bce * vf)
    cnt = jnp.sum(vf)

    x = x_ref[...]
    ct = ct_ref[0, 0, :]

    ones_m = jnp.full((_CM, _CM), 1.0, jnp.float32)
    dn = (((1,), (0,)), ((), ()))
    inv = 1.0 / _CM
    mu = jax.lax.dot_general(x, ones_m, dn,
                             preferred_element_type=jnp.float32) * inv
    ex2 = jax.lax.dot_general(x * x, ones_m, dn,
                              preferred_element_type=jnp.float32) * inv
    var = ex2 - mu * mu
    rs = jax.lax.rsqrt(var + 1e-5)
    normed = (x - mu) * rs * g_ref[0][None, :] + bb_ref[0][None, :]
    es = jnp.exp(normed)
    s3 = jax.lax.dot_general(es, ones_m, dn,
                             preferred_element_type=jnp.float32)
    logs3 = jnp.log(s3)

    lane = jax.lax.broadcasted_iota(jnp.int32, (_ROWS, _CM), 1)
    ohf = (lane == ct[:, None]).astype(jnp.float32)
    ohx = ohf * x
    nll2_sum = jnp.sum(ohf * (logs3 - normed))
    sum_xsel = jnp.sum(ohx)
    sum_xsel2 = jnp.sum(ohx * x)
    ccnt = jnp.sum(ohf)
    ppd_sum = ccnt - 2.0 * sum_xsel + sum_xsel2

    @pl.when(i == 0)
    def _():
        out_ref[0] = 0.0
        out_ref[1] = 0.0
        out_ref[2] = 0.0
        out_ref[3] = 0.0
        out_ref[4] = 0.0
        out_ref[5] = 0.0

    out_ref[0] += nll_sum
    out_ref[1] += bce_sum
    out_ref[2] += cnt
    out_ref[3] += nll2_sum
    out_ref[4] += ppd_sum
    out_ref[5] += ccnt


def kernel(seg, confidence, contrast_logits, contrast_target, target, ln_gamma, ln_beta):
    n = contrast_target.shape[0]
    ct3 = contrast_target.reshape(_STEPS, 1, n // _STEPS)
    g2 = ln_gamma.reshape(1, _CM)
    bb2 = ln_beta.reshape(1, _CM)

    sums = pl.pallas_call(
        _body,
        grid=(_STEPS,),
        in_specs=[
            pl.BlockSpec((1, _NC, _HCHUNK, 128), lambda i: (i // _SPB, 0, i % _SPB, 0)),
            pl.BlockSpec((1, _HCHUNK, 128), lambda i: (i // _SPB, i % _SPB, 0)),
            pl.BlockSpec((1, _HCHUNK, 128), lambda i: (i // _SPB, i % _SPB, 0)),
            pl.BlockSpec((_ROWS, _CM), lambda i: (i, 0)),
            pl.BlockSpec((1, 1, _ROWS), lambda i: (i, 0, 0)),
            pl.BlockSpec((1, _CM), lambda i: (0, 0)),
            pl.BlockSpec((1, _CM), lambda i: (0, 0)),
        ],
        out_specs=pl.BlockSpec(memory_space=pltpu.SMEM),
        out_shape=jax.ShapeDtypeStruct((6,), jnp.float32),
    )(seg, target, confidence, contrast_logits, ct3, g2, bb2)

    nll_sum, bce_sum, cnt, nll2_sum, ppd_sum, ccnt = (
        sums[0], sums[1], sums[2], sums[3], sums[4], sums[5])
    seg_loss = nll_sum / jnp.maximum(cnt, 1.0)
    uncer = bce_sum / jnp.maximum(cnt, 1.0)
    ppc = nll2_sum / jnp.maximum(ccnt, 1.0)
    ppd = ppd_sum / jnp.maximum(ccnt, 1.0)
    return seg_loss + _PPC_W * ppc + _PPD_W * ppd + _UNCER_W * uncer


# speedup vs baseline: 5.4628x; 1.0521x over previous
import jax
import jax.numpy as jnp
from jax.experimental import pallas as pl
from jax.experimental.pallas import tpu as pltpu

_NC = 19
_CM = 95
_IGNORE = -1
_PPC_W = 0.01
_PPD_W = 0.001
_UNCER_W = 1.0

_STEPS = 16
_ROWS = 131072 // _STEPS
_SPB = _STEPS // 8
_HCHUNK = 128 // _SPB


def _body(seg_ref, tgt_ref, conf_ref, x_ref, ct_ref, g_ref, bb_ref, out_ref):
    i = pl.program_id(0)

    seg = seg_ref[0]
    tgt = tgt_ref[0]
    conf = conf_ref[0]
    valid = tgt != _IGNORE
    vf = valid.astype(jnp.float32)
    tc = jnp.clip(tgt, 0, _NC - 1)

    m = jnp.max(seg, axis=0)
    cls_iota = jax.lax.broadcasted_iota(jnp.int32, seg.shape, 0)
    amax = jnp.min(jnp.where(seg == m[None], cls_iota, _NC), axis=0)
    seg2 = jnp.where(cls_iota == amax[None], -jnp.inf, seg)
    m2 = jnp.max(seg2, axis=0)

    s = jnp.sum(jnp.exp(seg - m[None]), axis=0)
    lse = m + jnp.log(s)
    seg_t = jnp.sum(jnp.where(cls_iota == tc[None], seg, 0.0), axis=0)
    nll_sum = jnp.sum((lse - seg_t) * vf)

    label = amax == tgt
    p = 1.0 / (1.0 + jnp.exp(m2 - m))
    u = jnp.where(label, 1.0 - p, p)
    bce = jnp.maximum(conf, 0.0) - conf * u + jnp.log1p(jnp.exp(-jnp.abs(conf)))
    bce_sum = jnp.sum(bce * vf)
    cnt = jnp.sum(vf)

    x = x_ref[...]
    ct = ct_ref[0, 0, :]
    ccnt = jnp.sum((ct != _IGNORE).astype(jnp.float32))

    ones_i = jnp.full((_CM, _CM), 1.0 / _CM, jnp.float32)
    ones_m = jnp.full((_CM, _CM), 1.0, jnp.float32)
    dn = (((1,), (0,)), ((), ()))
    mu = jax.lax.dot_general(x, ones_i, dn,
                             preferred_element_type=jnp.float32)
    ex2 = jax.lax.dot_general(x * x, ones_i, dn,
                              preferred_element_type=jnp.float32)
    var = ex2 - mu * mu
    rs = jax.lax.rsqrt(var + 1e-5)
    normed = (x - mu) * rs * g_ref[0][None, :] + bb_ref[0][None, :]
    es = jnp.exp(normed)
    s3 = jax.lax.dot_general(es, ones_m, dn,
                             preferred_element_type=jnp.float32)
    l2s3 = jnp.log2(s3)

    lane = jax.lax.broadcasted_iota(jnp.int32, (_ROWS, _CM), 1)
    oh = lane == ct[:, None]
    xm1 = x - 1.0
    c1 = _PPC_W * 0.6931471805599453
    t = c1 * l2s3 - _PPC_W * normed + _PPD_W * (xm1 * xm1)
    combo = jnp.sum(jnp.where(oh, t, 0.0))

    @pl.when(i == 0)
    def _():
        out_ref[0] = 0.0
        out_ref[1] = 0.0
        out_ref[2] = 0.0
        out_ref[3] = 0.0
        out_ref[4] = 0.0

    out_ref[0] += nll_sum
    out_ref[1] += bce_sum
    out_ref[2] += cnt
    out_ref[3] += combo
    out_ref[4] += ccnt


def kernel(seg, confidence, contrast_logits, contrast_target, target, ln_gamma, ln_beta):
    n = contrast_target.shape[0]
    ct3 = contrast_target.reshape(_STEPS, 1, n // _STEPS)
    g2 = ln_gamma.reshape(1, _CM)
    bb2 = ln_beta.reshape(1, _CM)

    sums = pl.pallas_call(
        _body,
        grid=(_STEPS,),
        in_specs=[
            pl.BlockSpec((1, _NC, _HCHUNK, 128), lambda i: (i // _SPB, 0, i % _SPB, 0)),
            pl.BlockSpec((1, _HCHUNK, 128), lambda i: (i // _SPB, i % _SPB, 0)),
            pl.BlockSpec((1, _HCHUNK, 128), lambda i: (i // _SPB, i % _SPB, 0)),
            pl.BlockSpec((_ROWS, _CM), lambda i: (i, 0)),
            pl.BlockSpec((1, 1, _ROWS), lambda i: (i, 0, 0)),
            pl.BlockSpec((1, _CM), lambda i: (0, 0)),
            pl.BlockSpec((1, _CM), lambda i: (0, 0)),
        ],
        out_specs=pl.BlockSpec(memory_space=pltpu.SMEM),
        out_shape=jax.ShapeDtypeStruct((5,), jnp.float32),
    )(seg, target, confidence, contrast_logits, ct3, g2, bb2)

    nll_sum, bce_sum, cnt, combo, ccnt = (
        sums[0], sums[1], sums[2], sums[3], sums[4])
    seg_loss = nll_sum / jnp.maximum(cnt, 1.0)
    uncer = bce_sum / jnp.maximum(cnt, 1.0)
    contrast = combo / jnp.maximum(ccnt, 1.0)
    return seg_loss + contrast + _UNCER_W * uncer
